# baseline (device time: 22886 ns/iter reference)
import functools

import jax
import jax.numpy as jnp
from jax import lax
from jax.experimental import pallas as pl
from jax.experimental.pallas import tpu as pltpu

N_DEV = 4
H_GLOBAL = 512
N_CHUNKS = 4


def kernel(x, Wp):
    b, h_per, w, c = x.shape
    c_out = Wp.shape[1]
    n_norm = float(H_GLOBAL * w)
    ch = h_per // N_CHUNKS

    xt = jnp.transpose(x, (0, 1, 3, 2))

    def body(x_hbm, wp_ref, out_hbm, xbuf, obuf, local_ref, stats_ref,
             in_sems, out_sems, send_sems, recv_sems):
        my = lax.axis_index("i")

        in_dmas = []
        for k in range(N_CHUNKS):
            dma = pltpu.make_async_copy(
                x_hbm.at[:, pl.ds(k * ch, ch)],
                xbuf.at[:, pl.ds(k * ch, ch)],
                in_sems.at[k],
            )
            dma.start()
            in_dmas.append(dma)

        barrier_sem = pltpu.get_barrier_semaphore()
        for off in (1, 2, 3):
            pl.semaphore_signal(
                barrier_sem, inc=1,
                device_id=((my + off) % N_DEV,),
                device_id_type=pl.DeviceIdType.MESH,
            )
        pl.semaphore_wait(barrier_sem, N_DEV - 1)

        ps = jnp.zeros((b, c), jnp.float32)
        pss = jnp.zeros((b, c), jnp.float32)
        for k in range(N_CHUNKS):
            in_dmas[k].wait()
            xk = xbuf[:, k * ch:(k + 1) * ch]
            ps = ps + jnp.sum(xk, axis=(1, 3))
            pss = pss + jnp.sum(xk * xk, axis=(1, 3))
        local_ref[...] = jnp.concatenate([ps, pss], axis=0)

        rdmas = []
        for off in (1, 2, 3):
            rdma = pltpu.make_async_remote_copy(
                src_ref=local_ref,
                dst_ref=stats_ref.at[off - 1],
                send_sem=send_sems.at[off - 1],
                recv_sem=recv_sems.at[off - 1],
                device_id=((my + off) % N_DEV,),
                device_id_type=pl.DeviceIdType.MESH,
            )
            rdma.start()
            rdmas.append(rdma)
        for rdma in rdmas:
            rdma.wait_recv()

        tot = local_ref[...] + stats_ref[0] + stats_ref[1] + stats_ref[2]
        mean = tot[:b, :] / n_norm
        var = tot[b:, :] / n_norm - mean * mean
        inv = lax.rsqrt(var + 1e-5)
        mb = mean[:, None, :, None]
        ib = inv[:, None, :, None]
        wb = wp_ref[...].astype(jnp.bfloat16)

        out_dmas = [None, None]
        for k in range(N_CHUNKS):
            slot = k % 2
            xk = xbuf[:, k * ch:(k + 1) * ch]
            hn = (xk - mb) * ib
            a = hn * jax.nn.sigmoid(hn)
            a2 = a.astype(jnp.bfloat16)
            o = lax.dot_general(
                a2, wb,
                dimension_numbers=(((2,), (0,)), ((), ())),
                preferred_element_type=jnp.float32,
            )
            if out_dmas[slot] is not None:
                out_dmas[slot].wait()
            obuf[slot] = o.astype(jnp.bfloat16)
            dma = pltpu.make_async_copy(
                obuf.at[slot],
                out_hbm.at[:, pl.ds(k * ch, ch)],
                out_sems.at[slot],
            )
            dma.start()
            out_dmas[slot] = dma
        for dma in out_dmas:
            dma.wait()
        for rdma in rdmas:
            rdma.wait_send()

        @functools.partial(
            pl.run_scoped, exit_sem=pltpu.SemaphoreType.REGULAR
        )
        def _(exit_sem):
            for off in (1, 2, 3):
                pl.semaphore_signal(
                    exit_sem, inc=1,
                    device_id=((my + off) % N_DEV,),
                    device_id_type=pl.DeviceIdType.MESH,
                )
            pl.semaphore_wait(exit_sem, N_DEV - 1)

    return pl.pallas_call(
        body,
        out_shape=jax.ShapeDtypeStruct((b, h_per, w, c_out), jnp.bfloat16),
        in_specs=[
            pl.BlockSpec(memory_space=pl.ANY),
            pl.BlockSpec(memory_space=pltpu.VMEM),
        ],
        out_specs=pl.BlockSpec(memory_space=pl.ANY),
        scratch_shapes=[
            pltpu.VMEM((b, h_per, c, w), jnp.float32),
            pltpu.VMEM((2, b, ch, w, c_out), jnp.bfloat16),
            pltpu.VMEM((2 * b, c), jnp.float32),
            pltpu.VMEM((N_DEV - 1, 2 * b, c), jnp.float32),
            pltpu.SemaphoreType.DMA((N_CHUNKS,)),
            pltpu.SemaphoreType.DMA((2,)),
            pltpu.SemaphoreType.DMA((N_DEV - 1,)),
            pltpu.SemaphoreType.DMA((N_DEV - 1,)),
        ],
        compiler_params=pltpu.CompilerParams(collective_id=0),
    )(xt, Wp)


# device time: 20730 ns/iter; 1.1040x vs baseline; 1.1040x over previous
import functools

import jax
import jax.numpy as jnp
from jax import lax
from jax.experimental import pallas as pl
from jax.experimental.pallas import tpu as pltpu

N_DEV = 4
H_GLOBAL = 512


def kernel(x, Wp):
    b, h_per, w, c = x.shape
    c_out = Wp.shape[1]
    n_norm = float(H_GLOBAL * w)

    xt = jnp.transpose(x, (0, 1, 3, 2))

    def body(x_ref, wp_ref, out_ref, xb_ref, local_ref, stats_ref,
             send_sems, recv_sems):
        my = lax.axis_index("i")

        barrier_sem = pltpu.get_barrier_semaphore()
        for off in (1, 2, 3):
            pl.semaphore_signal(
                barrier_sem, inc=1,
                device_id=((my + off) % N_DEV,),
                device_id_type=pl.DeviceIdType.MESH,
            )
        pl.semaphore_wait(barrier_sem, N_DEV - 1)

        xv = x_ref[...]
        ps = jnp.sum(xv, axis=(1, 3))
        pss = jnp.sum(xv * xv, axis=(1, 3))
        local_ref[...] = jnp.concatenate([ps, pss], axis=0)

        rdmas = []
        for off in (1, 2, 3):
            rdma = pltpu.make_async_remote_copy(
                src_ref=local_ref,
                dst_ref=stats_ref.at[off - 1],
                send_sem=send_sems.at[off - 1],
                recv_sem=recv_sems.at[off - 1],
                device_id=((my + off) % N_DEV,),
                device_id_type=pl.DeviceIdType.MESH,
            )
            rdma.start()
            rdmas.append(rdma)

        xb_ref[...] = xv.astype(jnp.bfloat16)

        for rdma in rdmas:
            rdma.wait_recv()

        tot = local_ref[...] + stats_ref[0] + stats_ref[1] + stats_ref[2]
        mean = tot[:b, :] / n_norm
        var = tot[b:, :] / n_norm - mean * mean
        inv = lax.rsqrt(var + 1e-5)
        mb = mean.astype(jnp.bfloat16)[:, None, :, None]
        ib = inv.astype(jnp.bfloat16)[:, None, :, None]

        hn = (xb_ref[...] - mb) * ib
        a = hn * jax.nn.sigmoid(hn)
        o = lax.dot_general(
            a, wp_ref[...].astype(jnp.bfloat16),
            dimension_numbers=(((2,), (0,)), ((), ())),
            preferred_element_type=jnp.float32,
        )
        out_ref[...] = o.astype(jnp.bfloat16)

        for rdma in rdmas:
            rdma.wait_send()

        @functools.partial(
            pl.run_scoped, exit_sem=pltpu.SemaphoreType.REGULAR
        )
        def _(exit_sem):
            for off in (1, 2, 3):
                pl.semaphore_signal(
                    exit_sem, inc=1,
                    device_id=((my + off) % N_DEV,),
                    device_id_type=pl.DeviceIdType.MESH,
                )
            pl.semaphore_wait(exit_sem, N_DEV - 1)

    return pl.pallas_call(
        body,
        out_shape=jax.ShapeDtypeStruct((b, h_per, w, c_out), jnp.bfloat16),
        in_specs=[
            pl.BlockSpec(memory_space=pltpu.VMEM),
            pl.BlockSpec(memory_space=pltpu.VMEM),
        ],
        out_specs=pl.BlockSpec(memory_space=pltpu.VMEM),
        scratch_shapes=[
            pltpu.VMEM((b, h_per, c, w), jnp.bfloat16),
            pltpu.VMEM((2 * b, c), jnp.float32),
            pltpu.VMEM((N_DEV - 1, 2 * b, c), jnp.float32),
            pltpu.SemaphoreType.DMA((N_DEV - 1,)),
            pltpu.SemaphoreType.DMA((N_DEV - 1,)),
        ],
        compiler_params=pltpu.CompilerParams(collective_id=0),
    )(xt, Wp)
